# Initial kernel scaffold; baseline (speedup 1.0000x reference)
import jax
import jax.numpy as jnp
from jax.experimental import pallas as pl


def kernel(x, edge_index, batch, W1, a_src1, a_dst1, b1, W2, a_src2, a_dst2, b2, Wp, bp, Wc, bc):
    raise NotImplementedError("write your pallas kernel here")


# speedup vs baseline: 1.0017x; 1.0017x over previous
import functools

import jax
import jax.numpy as jnp
from jax.experimental import pallas as pl
from jax.experimental.pallas import tpu as pltpu

N = 10000
B = 64
HID = 128
POIN = 32
NCLS = 7


def _pool_head_kernel(h_ref, batch_ref, Wp_ref, bp_ref, Wc_ref, bc_ref,
                      logits_ref, z_ref):
    h = h_ref[...]
    batch = batch_ref[...]
    onehot = (batch == jax.lax.broadcasted_iota(jnp.int32, (1, B), 1)).astype(jnp.float32)
    sums = jax.lax.dot_general(onehot, h, (((0,), (0,)), ((), ())))
    counts = jnp.sum(onehot, axis=0, keepdims=True)
    pooled = sums / jnp.maximum(counts.T, 1.0)
    z = pooled @ Wp_ref[...] + bp_ref[...]
    norm = jnp.sqrt(jnp.sum(z * z, axis=1, keepdims=True))
    max_norm = 1.0 - 1e-05
    z = jnp.where(norm >= max_norm, z / jnp.maximum(norm, 1e-12) * max_norm, z)
    logits = z @ Wc_ref[...] + bc_ref[...]
    logits_ref[...] = logits
    z_ref[...] = z


def _pool_head(h, batch, Wp, bp, Wc, bc):
    return pl.pallas_call(
        _pool_head_kernel,
        out_shape=(jax.ShapeDtypeStruct((B, NCLS), jnp.float32),
                   jax.ShapeDtypeStruct((B, POIN), jnp.float32)),
    )(h, batch.reshape(N, 1), Wp, bp.reshape(1, POIN), Wc, bc.reshape(1, NCLS))


def _gat_conv(x, edge_index, W, a_src, a_dst, bias, heads, out_ch, concat):
    n = x.shape[0]
    loops = jnp.arange(n, dtype=edge_index.dtype)
    src = jnp.concatenate([edge_index[0], loops])
    dst = jnp.concatenate([edge_index[1], loops])
    h = (x @ W).reshape(n, heads, out_ch)
    alpha_src = jnp.sum(h * a_src, axis=-1)
    alpha_dst = jnp.sum(h * a_dst, axis=-1)
    e = alpha_src[src] + alpha_dst[dst]
    e = jax.nn.leaky_relu(e, negative_slope=0.2)
    e_max = jax.ops.segment_max(e, dst, num_segments=n)
    e_exp = jnp.exp(e - e_max[dst])
    denom = jax.ops.segment_sum(e_exp, dst, num_segments=n)
    alpha = e_exp / (denom[dst] + 1e-16)
    msg = h[src] * alpha[:, :, None]
    out = jax.ops.segment_sum(msg, dst, num_segments=n)
    if concat:
        out = out.reshape(n, heads * out_ch)
    else:
        out = out.mean(axis=1)
    return out + bias


def kernel(x, edge_index, batch, W1, a_src1, a_dst1, b1, W2, a_src2, a_dst2, b2, Wp, bp, Wc, bc):
    h = jax.nn.elu(_gat_conv(x, edge_index, W1, a_src1, a_dst1, b1, 8, HID, True))
    h = jax.nn.elu(_gat_conv(h, edge_index, W2, a_src2, a_dst2, b2, 1, HID, False))
    return _pool_head(h, batch, Wp, bp, Wc, bc)


# speedup vs baseline: 24.2177x; 24.1757x over previous
import jax
import jax.numpy as jnp
from jax import lax
from jax.experimental import pallas as pl
from jax.experimental.pallas import tpu as pltpu
from jax.experimental.pallas import tpu_sc as plsc

N = 10000
E = 320000
IN_DIM = 128
HID = 128
HEADS = 8
POIN = 32
NCLS = 7
B = 64

ROW1 = 1040
ROW2 = 144
RANGE1 = 1280
SLAB1 = 80
JUNK1 = 1280
ACC1_ROWS = 1288
RANGE2 = 5000
SLAB2 = 320
JUNK2 = 5000
ACC2_ROWS = 5008
ETILE = E // 16
CHUNK = 2000
NCHUNK = ETILE // CHUNK
FBLK = CHUNK // 16
ROWBLK = 400
GRID = N // ROWBLK

_MESH = plsc.VectorSubcoreMesh(core_axis_name="c", subcore_axis_name="s")


def _tc1a_body(x_ref, W1_ref, As_ref, Ad_ref, tab_ref, adtab_ref, pmax_ref):
    h = jnp.dot(x_ref[...], W1_ref[...], preferred_element_type=jnp.float32)
    asrc = jnp.dot(h, As_ref[...], preferred_element_type=jnp.float32)
    adst = jnp.dot(h, Ad_ref[...], preferred_element_type=jnp.float32)
    z8 = jnp.zeros((ROWBLK, 8), jnp.float32)
    tab_ref[...] = jnp.concatenate([h, asrc, z8], axis=1)
    adtab_ref[...] = jnp.concatenate([adst, z8], axis=1)
    pmax_ref[...] = jnp.concatenate(
        [jnp.max(asrc, axis=0, keepdims=True),
         jnp.max(adst, axis=0, keepdims=True)], axis=1).reshape(1, 1, 16)


def _tc1c_body(acc_ref, b1_ref, W2_ref, a2s_ref, a2d_ref,
               tab2_ref, adtab2_ref, pmax2_ref):
    acc = acc_ref[...]
    den = acc[:, 1024:1032]
    denrep = jnp.reshape(jnp.broadcast_to(den[:, :, None], (ROWBLK, 8, 128)),
                         (ROWBLK, 1024))
    h2in = acc[:, :1024] / denrep + b1_ref[...]
    h2in = jnp.where(h2in > 0, h2in, jnp.exp(h2in) - 1.0)
    h2 = jnp.dot(h2in, W2_ref[...], preferred_element_type=jnp.float32)
    s2 = jnp.dot(h2, a2s_ref[...], preferred_element_type=jnp.float32)
    d2 = jnp.dot(h2, a2d_ref[...], preferred_element_type=jnp.float32)
    tab2_ref[...] = jnp.concatenate(
        [h2, s2, jnp.zeros((ROWBLK, 15), jnp.float32)], axis=1)
    adtab2_ref[...] = jnp.concatenate(
        [d2, jnp.zeros((ROWBLK, 15), jnp.float32)], axis=1)
    pmax2_ref[...] = jnp.concatenate(
        [jnp.max(s2, axis=0, keepdims=True),
         jnp.zeros((1, 7), jnp.float32),
         jnp.max(d2, axis=0, keepdims=True),
         jnp.zeros((1, 7), jnp.float32)], axis=1).reshape(1, 1, 16)


def _final_body(acc_ref, batch_ref, b2_ref, Wp_ref, bp_ref, Wc_ref, bc_ref,
                logits_ref, z_ref):
    acc = acc_ref[...]
    h3 = acc[:, :128] / acc[:, 128:129] + b2_ref[...]
    h3 = jnp.where(h3 > 0, h3, jnp.exp(h3) - 1.0)
    onehot = (batch_ref[...] == lax.broadcasted_iota(jnp.int32, (1, B), 1)
              ).astype(jnp.float32)
    sums = lax.dot_general(onehot, h3, (((0,), (0,)), ((), ())),
                           preferred_element_type=jnp.float32)
    counts = jnp.sum(onehot, axis=0, keepdims=True)
    pooled = sums / jnp.maximum(counts.T, 1.0)
    z = pooled @ Wp_ref[...] + bp_ref[...]
    norm = jnp.sqrt(jnp.sum(z * z, axis=1, keepdims=True))
    max_norm = 1.0 - 1e-05
    z = jnp.where(norm >= max_norm, z / jnp.maximum(norm, 1e-12) * max_norm, z)
    logits_ref[...] = z @ Wc_ref[...] + bc_ref[...]
    z_ref[...] = z


def _prefix16(m, lane, psbuf):
    v = jnp.where(m, 1.0, 0.0)
    zeros = jnp.zeros((16,), jnp.float32)
    for k in (1, 2, 4, 8):
        psbuf[pl.ds(16, 16)] = zeros
        plsc.store_scatter(psbuf, [lane + (16 + k)], v)
        v = v + psbuf[pl.ds(16, 16)]
    return v.astype(jnp.int32)


def _load_gmax(pmax, pbuf, gbuf):
    pltpu.sync_copy(pmax, pbuf)

    def red(i, mx):
        return jnp.maximum(mx, pbuf[pl.ds(i * 16, 16)])

    mx = lax.fori_loop(0, GRID, red, jnp.full((16,), -3e38, jnp.float32))
    gbuf[pl.ds(16, 16)] = jnp.zeros((16,), jnp.float32)
    gbuf[pl.ds(0, 16)] = mx
    g = mx + gbuf[pl.ds(8, 16)]
    return jnp.where(g >= 0, g, 0.2 * g)


def _edge_pipeline(nb, tab, adtab, srclist, dstlist, dglist, accsh,
                   rowsA, rowsB, adstbA, adstbB,
                   semrA, semrB, semaA, semaB, semsA, semsB, compute16, eb):
    nsub = eb // 16

    def start_gather(b, rbuf, abuf, semr, sema):
        pltpu.async_copy(tab.at[srclist.at[pl.ds(b * eb, eb)]], rbuf, semr)
        pltpu.async_copy(adtab.at[dglist.at[pl.ds(b * eb, eb)]], abuf, sema)

    @pl.when(nb > 0)
    def _():
        start_gather(0, rowsA, adstbA, semrA, semaA)

    def do_batch(b, rbuf, abuf, semr, sema, sems, nrbuf, nabuf,
                 nsemr, nsema, nsems):
        pltpu.make_async_copy(tab.at[srclist.at[pl.ds(b * eb, eb)]],
                              rbuf, semr).wait()
        pltpu.make_async_copy(adtab.at[dglist.at[pl.ds(b * eb, eb)]],
                              abuf, sema).wait()

        @pl.when(b >= 1)
        def _():
            for k in range(nsub):
                dlvk = dstlist[pl.ds((b - 1) * eb + k * 16, 16)]
                pltpu.make_async_copy(nrbuf.at[pl.ds(k * 16, 16)],
                                      accsh.at[dlvk], nsems).wait()

        @pl.when(b + 1 < nb)
        def _():
            start_gather(b + 1, nrbuf, nabuf, nsemr, nsema)

        for k in range(nsub):
            compute16(rbuf, abuf, k)
            dlvk = dstlist[pl.ds(b * eb + k * 16, 16)]
            pltpu.async_copy(rbuf.at[pl.ds(k * 16, 16)], accsh.at[dlvk],
                             sems, add=True)

    def pair(g, _):
        b0 = 2 * g

        @pl.when(b0 < nb)
        def _():
            do_batch(b0, rowsA, adstbA, semrA, semaA, semsA,
                     rowsB, adstbB, semrB, semaB, semsB)

        @pl.when(b0 + 1 < nb)
        def _():
            do_batch(b0 + 1, rowsB, adstbB, semrB, semaB, semsB,
                     rowsA, adstbA, semrA, semaA, semsA)

        return 0

    lax.fori_loop(0, (nb + 1) // 2, pair, 0)
    dlv0 = dstlist[pl.ds(0, 16)]

    @pl.when(nb % 2 == 1)
    def _():
        for k in range(nsub):
            pltpu.make_async_copy(rowsA.at[pl.ds(k * 16, 16)],
                                  accsh.at[dlv0], semsA).wait()

    @pl.when((nb >= 1) & (nb % 2 == 0))
    def _():
        for k in range(nsub):
            pltpu.make_async_copy(rowsB.at[pl.ds(k * 16, 16)],
                                  accsh.at[dlv0], semsB).wait()


def _scan_filter(s, lo, rng, junk, srch, dsth, srclist, dstlist, dglist,
                 srcch, dstch, lane, psbuf, c2, eb):
    base = s * ETILE + c2 * CHUNK
    pltpu.sync_copy(srch.at[pl.ds(base, CHUNK)], srcch)
    pltpu.sync_copy(dsth.at[pl.ds(base, CHUNK)], dstch)

    def filt(f, cnt):
        dv = dstch[pl.ds(f * 16, 16)]
        sv = srcch[pl.ds(f * 16, 16)]
        m = (dv >= lo) & (dv < lo + rng)
        csum = _prefix16(m, lane, psbuf)
        pos = cnt + csum - 1
        plsc.store_scatter(dstlist, [pos], dv - lo, mask=m)
        plsc.store_scatter(srclist, [pos], sv, mask=m)
        plsc.store_scatter(dglist, [pos], dv, mask=m)
        return cnt + csum[15]

    cnt = lax.fori_loop(0, FBLK, filt, jnp.int32(0))
    for j in range(eb // 16):
        dstlist[pl.ds(cnt + j * 16, 16)] = jnp.full((16,), junk, jnp.int32)
        srclist[pl.ds(cnt + j * 16, 16)] = jnp.zeros((16,), jnp.int32)
        dglist[pl.ds(cnt + j * 16, 16)] = jnp.full((16,), N - 1, jnp.int32)
    return (cnt + eb - 1) // eb


def _sc1_body(tab, adtab, srch, dsth, pmax, acc_out,
              srclist, dstlist, dglist, srcch, dstch, rowsA, rowsB,
              adstbA, adstbB, gbuf, psbuf, pbuf,
              semrA, semrB, semaA, semaB, semsA, semsB, accsh):
    c = lax.axis_index("c")
    s = lax.axis_index("s")
    gv = _load_gmax(pmax, pbuf, gbuf)
    lane = lax.broadcasted_iota(jnp.int32, (16,), 0)
    headmask = lane < HEADS

    def compute16(rbuf, abuf, k):
        def edge(i0, _):
            i = k * 16 + i0
            av = rbuf[i, pl.ds(1024, 16)]
            bv = abuf[i, :]
            e = av + bv
            e = jnp.where(e >= 0, e, 0.2 * e)
            w = jnp.exp(e - gv)
            w = jnp.where(headmask, w, 0.0)
            for hd in range(HEADS):
                ws = w[hd]
                for j in range(8):
                    col = hd * 128 + j * 16
                    rbuf[i, pl.ds(col, 16)] = rbuf[i, pl.ds(col, 16)] * ws
            rbuf[i, pl.ds(1024, 16)] = w
            return 0

        lax.fori_loop(0, 16, edge, 0)

    def pass_body(p, _):
        lo = (c * 4 + p) * RANGE1
        gs = jnp.minimum(lo + s * SLAB1, N - SLAB1)
        off = gs - lo

        def init_slab(k, _):
            pltpu.sync_copy(tab.at[pl.ds(gs + k * 16, 16)], rowsA)
            pltpu.sync_copy(adtab.at[pl.ds(gs + k * 16, 16)], adstbA)
            compute16(rowsA, adstbA, 0)
            pltpu.sync_copy(rowsA, accsh.at[pl.ds(off + k * 16, 16)])
            return 0

        lax.fori_loop(0, SLAB1 // 16, init_slab, 0)
        plsc.subcore_barrier()

        def chunk_body(c2, _):
            nb = _scan_filter(s, lo, RANGE1, JUNK1, srch, dsth, srclist,
                              dstlist, dglist, srcch, dstch, lane, psbuf,
                              c2, 16)
            _edge_pipeline(nb, tab, adtab, srclist, dstlist, dglist, accsh,
                           rowsA, rowsB, adstbA, adstbB,
                           semrA, semrB, semaA, semaB, semsA, semsB,
                           compute16, 16)
            return 0

        lax.fori_loop(0, NCHUNK, chunk_body, 0)
        plsc.subcore_barrier()
        pltpu.sync_copy(accsh.at[pl.ds(off, SLAB1)],
                        acc_out.at[pl.ds(gs, SLAB1)])
        plsc.subcore_barrier()
        return 0

    lax.fori_loop(0, 4, pass_body, 0)


def _sc2_body(tab, adtab, srch, dsth, pmax, acc_out,
              srclist, dstlist, dglist, srcch, dstch, rowsA, rowsB,
              adstbA, adstbB, gbuf, psbuf, pbuf,
              semrA, semrB, semaA, semaB, semsA, semsB, accsh):
    c = lax.axis_index("c")
    s = lax.axis_index("s")
    g2 = _load_gmax(pmax, pbuf, gbuf)[0]
    lane = lax.broadcasted_iota(jnp.int32, (16,), 0)

    def compute16(rbuf, abuf, k):
        def edge(i0, _):
            i = k * 16 + i0
            va = rbuf[i, pl.ds(128, 16)]
            vb = abuf[i, :]
            ev = va + vb
            ev = jnp.where(ev >= 0, ev, 0.2 * ev)
            wv = jnp.exp(ev - g2)
            ws = wv[0]
            for j in range(8):
                rbuf[i, pl.ds(j * 16, 16)] = rbuf[i, pl.ds(j * 16, 16)] * ws
            rbuf[i, pl.ds(128, 16)] = jnp.where(lane == 0, ws, 0.0)
            return 0

        lax.fori_loop(0, 16, edge, 0)

    lo = c * RANGE2
    gs = lo + jnp.minimum(s * SLAB2, RANGE2 - SLAB2)
    off = gs - lo

    def init_slab(k, _):
        pltpu.sync_copy(tab.at[pl.ds(gs + k * 64, 64)], rowsA)
        pltpu.sync_copy(adtab.at[pl.ds(gs + k * 64, 64)], adstbA)
        for kk in range(4):
            compute16(rowsA, adstbA, kk)
        pltpu.sync_copy(rowsA, accsh.at[pl.ds(off + k * 64, 64)])
        return 0

    lax.fori_loop(0, SLAB2 // 64, init_slab, 0)
    plsc.subcore_barrier()

    def chunk_body(c2, _):
        nb = _scan_filter(s, lo, RANGE2, JUNK2, srch, dsth, srclist,
                          dstlist, dglist, srcch, dstch, lane, psbuf,
                          c2, 64)
        _edge_pipeline(nb, tab, adtab, srclist, dstlist, dglist, accsh,
                       rowsA, rowsB, adstbA, adstbB,
                       semrA, semrB, semaA, semaB, semsA, semsB,
                       compute16, 64)
        return 0

    lax.fori_loop(0, NCHUNK, chunk_body, 0)
    plsc.subcore_barrier()
    pltpu.sync_copy(accsh.at[pl.ds(off, SLAB2)], acc_out.at[pl.ds(gs, SLAB2)])
    plsc.subcore_barrier()


def _make_sc(body, row, acc_rows, eb):
    return pl.kernel(
        body,
        out_type=jax.ShapeDtypeStruct((N, row), jnp.float32),
        mesh=_MESH,
        compiler_params=pltpu.CompilerParams(needs_layout_passes=False,
                                             use_tc_tiling_on_sc=False),
        scratch_types=[
            pltpu.VMEM((CHUNK + 64,), jnp.int32),
            pltpu.VMEM((CHUNK + 64,), jnp.int32),
            pltpu.VMEM((CHUNK + 64,), jnp.int32),
            pltpu.VMEM((CHUNK,), jnp.int32),
            pltpu.VMEM((CHUNK,), jnp.int32),
            pltpu.VMEM((eb, row), jnp.float32),
            pltpu.VMEM((eb, row), jnp.float32),
            pltpu.VMEM((eb, 16), jnp.float32),
            pltpu.VMEM((eb, 16), jnp.float32),
            pltpu.VMEM((32,), jnp.float32),
            pltpu.VMEM((48,), jnp.float32),
            pltpu.VMEM((GRID * 16,), jnp.float32),
            pltpu.SemaphoreType.DMA,
            pltpu.SemaphoreType.DMA,
            pltpu.SemaphoreType.DMA,
            pltpu.SemaphoreType.DMA,
            pltpu.SemaphoreType.DMA,
            pltpu.SemaphoreType.DMA,
            pltpu.VMEM_SHARED((acc_rows, row), jnp.float32),
        ],
    )


def kernel(x, edge_index, batch, W1, a_src1, a_dst1, b1,
           W2, a_src2, a_dst2, b2, Wp, bp, Wc, bc):
    src = edge_index[0]
    dst = edge_index[1]
    eye8 = jnp.eye(HEADS, dtype=jnp.float32)
    Asrc = jnp.reshape(a_src1[0][:, :, None] * eye8[:, None, :],
                       (HEADS * HID, HEADS))
    Adst = jnp.reshape(a_dst1[0][:, :, None] * eye8[:, None, :],
                       (HEADS * HID, HEADS))

    tab1, adtab1, pmax1 = pl.pallas_call(
        _tc1a_body,
        grid=(GRID,),
        in_specs=[
            pl.BlockSpec((ROWBLK, IN_DIM), lambda i: (i, 0)),
            pl.BlockSpec((IN_DIM, HEADS * HID), lambda i: (0, 0)),
            pl.BlockSpec((HEADS * HID, HEADS), lambda i: (0, 0)),
            pl.BlockSpec((HEADS * HID, HEADS), lambda i: (0, 0)),
        ],
        out_specs=[
            pl.BlockSpec((ROWBLK, ROW1), lambda i: (i, 0)),
            pl.BlockSpec((ROWBLK, 16), lambda i: (i, 0)),
            pl.BlockSpec((1, 1, 16), lambda i: (i, 0, 0)),
        ],
        out_shape=[
            jax.ShapeDtypeStruct((N, ROW1), jnp.float32),
            jax.ShapeDtypeStruct((N, 16), jnp.float32),
            jax.ShapeDtypeStruct((GRID, 1, 16), jnp.float32),
        ],
    )(x, W1, Asrc, Adst)

    sc1 = _make_sc(_sc1_body, ROW1, ACC1_ROWS, 16)
    acc1 = sc1(tab1, adtab1, src, dst, pmax1.reshape(GRID * 16))

    tab2, adtab2, pmax2 = pl.pallas_call(
        _tc1c_body,
        grid=(GRID,),
        in_specs=[
            pl.BlockSpec((ROWBLK, ROW1), lambda i: (i, 0)),
            pl.BlockSpec((1, HEADS * HID), lambda i: (0, 0)),
            pl.BlockSpec((HEADS * HID, HID), lambda i: (0, 0)),
            pl.BlockSpec((HID, 1), lambda i: (0, 0)),
            pl.BlockSpec((HID, 1), lambda i: (0, 0)),
        ],
        out_specs=[
            pl.BlockSpec((ROWBLK, ROW2), lambda i: (i, 0)),
            pl.BlockSpec((ROWBLK, 16), lambda i: (i, 0)),
            pl.BlockSpec((1, 1, 16), lambda i: (i, 0, 0)),
        ],
        out_shape=[
            jax.ShapeDtypeStruct((N, ROW2), jnp.float32),
            jax.ShapeDtypeStruct((N, 16), jnp.float32),
            jax.ShapeDtypeStruct((GRID, 1, 16), jnp.float32),
        ],
    )(acc1, b1.reshape(1, HEADS * HID), W2,
      a_src2.reshape(HID, 1), a_dst2.reshape(HID, 1))

    sc2 = _make_sc(_sc2_body, ROW2, ACC2_ROWS, 64)
    acc2 = sc2(tab2, adtab2, src, dst, pmax2.reshape(GRID * 16))

    logits, z = pl.pallas_call(
        _final_body,
        out_shape=(jax.ShapeDtypeStruct((B, NCLS), jnp.float32),
                   jax.ShapeDtypeStruct((B, POIN), jnp.float32)),
    )(acc2, batch.reshape(N, 1), b2.reshape(1, HID), Wp,
      bp.reshape(1, POIN), Wc, bc.reshape(1, NCLS))
    return logits, z
